# baseline (speedup 1.0000x reference)
import jax
import jax.numpy as jnp
from jax.experimental import pallas as pl
from jax.experimental.pallas import tpu as pltpu

_N_CHUNKS = 8


def _copy_kernel(src, dst, sems):
    n = src.shape[0] // _N_CHUNKS
    copies = [
        pltpu.make_async_copy(
            src.at[pl.ds(i * n, n)], dst.at[pl.ds(i * n, n)], sems.at[i]
        )
        for i in range(_N_CHUNKS)
    ]
    for c in copies:
        c.start()
    for c in copies:
        c.wait()


def kernel(embedding):
    return pl.pallas_call(
        _copy_kernel,
        in_specs=[pl.BlockSpec(memory_space=pl.ANY)],
        out_specs=pl.BlockSpec(memory_space=pl.ANY),
        out_shape=jax.ShapeDtypeStruct(embedding.shape, embedding.dtype),
        scratch_shapes=[pltpu.SemaphoreType.DMA((_N_CHUNKS,))],
    )(embedding)


# speedup vs baseline: 3.7551x; 3.7551x over previous
import jax
import jax.numpy as jnp
from jax.experimental import pallas as pl
from jax.experimental.pallas import tpu as pltpu

_ROWS = 77000
_COLS = 512
_BLOCK_ROWS = 3080


def _copy_kernel(src, dst):
    dst[...] = src[...]


def kernel(embedding):
    flat = embedding.reshape(_ROWS, _COLS)
    out = pl.pallas_call(
        _copy_kernel,
        grid=(_ROWS // _BLOCK_ROWS,),
        in_specs=[pl.BlockSpec((_BLOCK_ROWS, _COLS), lambda i: (i, 0))],
        out_specs=pl.BlockSpec((_BLOCK_ROWS, _COLS), lambda i: (i, 0)),
        out_shape=jax.ShapeDtypeStruct((_ROWS, _COLS), embedding.dtype),
    )(flat)
    return out.reshape(embedding.shape)
